# Initial kernel scaffold; baseline (speedup 1.0000x reference)
import jax
import jax.numpy as jnp
from jax.experimental import pallas as pl


def kernel(pred_raw, u_c, theta_c, connectivity, elem_lengths, prop_E, prop_A, prop_I22, elem_directions, F_ext, bc_disp, bc_rot):
    raise NotImplementedError("write your pallas kernel here")


# speedup vs baseline: 1429.5222x; 1429.5222x over previous
import jax
import jax.numpy as jnp
from jax import lax
from jax.experimental import pallas as pl
from jax.experimental.pallas import tpu as pltpu
from jax.experimental.pallas import tpu_sc as plsc

NC = 2
NS = 16
NW = NC * NS
LANES = 16
CHUNK = 2000


def _sc_forces(n_nodes, n_elem):
    ew = n_elem // NW
    nchunk = ew // CHUNK
    groups = CHUNK // LANES
    npt = -(-n_nodes // NS // 128) * 128
    accn = NS * npt

    def body(ux, uz, uth, na, nb, el, pe, pa, pi, ecs, esn, uc16, th16, zsrc,
             out, *scr):
        i32 = jnp.int32
        cid = lax.axis_index("c")
        sid = lax.axis_index("s")
        wid = cid * i32(NS) + sid

        setA = scr[0:24]
        setB = scr[24:48]
        uc_v, th_v, zv, ax, az, at = scr[48:54]

        pltpu.sync_copy(zsrc, zv)
        sl0 = pl.ds(sid * i32(npt), npt)
        pltpu.sync_copy(zv, ax.at[sl0])
        pltpu.sync_copy(zv, az.at[sl0])
        pltpu.sync_copy(zv, at.at[sl0])
        pltpu.sync_copy(uc16, uc_v)
        pltpu.sync_copy(th16, th_v)
        plsc.subcore_barrier()
        uc = uc_v[...]
        th = th_v[...]

        def unpack(S):
            return dict(na_v=S[0], nb_v=S[1], l_v=S[2], e_v=S[3], a_v=S[4],
                        i_v=S[5], cs_v=S[6], sn_v=S[7],
                        g=S[8:14], f=S[14:20],
                        sem_n=S[20], sem_l=S[21], sem_g=S[22], sem_s=S[23])

        def start_idx_loads(S, k):
            base = wid * i32(ew) + k * i32(CHUNK)
            pltpu.async_copy(na.at[pl.ds(base, CHUNK)], S["na_v"], S["sem_n"])
            pltpu.async_copy(nb.at[pl.ds(base, CHUNK)], S["nb_v"], S["sem_n"])

        def start_elem_loads(S, k):
            base = wid * i32(ew) + k * i32(CHUNK)
            for src, dst in ((el, S["l_v"]), (pe, S["e_v"]), (pa, S["a_v"]),
                             (pi, S["i_v"])):
                pltpu.async_copy(src.at[pl.ds(base, CHUNK)], dst, S["sem_l"])
            pltpu.async_copy(ecs.at[pl.ds(base, CHUNK)], S["cs_v"],
                             S["sem_l"])
            pltpu.async_copy(esn.at[pl.ds(base, CHUNK)], S["sn_v"],
                             S["sem_l"])

        def wait_idx_loads(S):
            base = pl.ds(0, CHUNK)
            pltpu.make_async_copy(na.at[base], S["na_v"], S["sem_n"]).wait()
            pltpu.make_async_copy(nb.at[base], S["nb_v"], S["sem_n"]).wait()

        def wait_elem_loads(S):
            base = pl.ds(0, CHUNK)
            for src, dst in ((el, S["l_v"]), (pe, S["e_v"]), (pa, S["a_v"]),
                             (pi, S["i_v"])):
                pltpu.make_async_copy(src.at[base], dst, S["sem_l"]).wait()
            pltpu.make_async_copy(ecs.at[base], S["cs_v"], S["sem_l"]).wait()
            pltpu.make_async_copy(esn.at[base], S["sn_v"], S["sem_l"]).wait()

        def gather_list(S):
            return ((ux, S["na_v"], S["g"][0]), (uz, S["na_v"], S["g"][1]),
                    (uth, S["na_v"], S["g"][2]), (ux, S["nb_v"], S["g"][3]),
                    (uz, S["nb_v"], S["g"][4]), (uth, S["nb_v"], S["g"][5]))

        def scatter_list(S):
            return ((S["f"][0], S["na_v"], ax), (S["f"][1], S["na_v"], az),
                    (S["f"][2], S["na_v"], at), (S["f"][3], S["nb_v"], ax),
                    (S["f"][4], S["nb_v"], az), (S["f"][5], S["nb_v"], at))

        def start_gathers(S):
            for tab, ixr, dst in gather_list(S):
                pltpu.async_copy(tab.at[ixr], dst, S["sem_g"])

        def wait_gathers(S):
            for tab, ixr, dst in gather_list(S):
                pltpu.make_async_copy(tab.at[ixr], dst, S["sem_g"]).wait()

        def start_scatters(S):
            for src, ixr, accr in scatter_list(S):
                pltpu.async_copy(src, accr.at[ixr], S["sem_s"], add=True)

        def wait_scatters(S):
            for src, ixr, accr in scatter_list(S):
                pltpu.make_async_copy(src, accr.at[ixr], S["sem_s"]).wait()

        def compute(S):
            gax, gaz, gat, gbx, gbz, gbt = S["g"]
            fax, faz, fat, fbx, fbz, fbt = S["f"]
            cs_v, sn_v, l_v, e_v, a_v, i_v = (S["cs_v"], S["sn_v"], S["l_v"],
                                              S["e_v"], S["a_v"], S["i_v"])

            @plsc.parallel_loop(jnp.int32(0), jnp.int32(groups), jnp.int32(1),
                                unroll=2)
            def group(g):
                sl = pl.ds(g * i32(LANES), LANES)
                uxA = gax[sl] * uc
                uzA = gaz[sl] * uc
                thA = gat[sl] * th
                uxB = gbx[sl] * uc
                uzB = gbz[sl] * uc
                thB = gbt[sl] * th
                cc = cs_v[sl]
                ss = sn_v[sl]
                rl = 1.0 / l_v[sl]
                eg = e_v[sl]
                ea_l = eg * a_v[sl] * rl
                ei_l = eg * i_v[sl] * rl
                ei_l2 = ei_l * rl
                ei_l3 = ei_l2 * rl
                dx = uxA - uxB
                dz = uzA - uzB
                du = cc * dx + ss * dz
                dw = cc * dz - ss * dx
                tA = -thA
                tB = -thB
                f0 = ea_l * du
                f1 = 12.0 * ei_l3 * dw + 6.0 * ei_l2 * (tA + tB)
                sdw = 6.0 * ei_l2 * dw
                f2 = sdw + ei_l * (4.0 * tA + 2.0 * tB)
                f5 = sdw + ei_l * (2.0 * tA + 4.0 * tB)
                fx = cc * f0 - ss * f1
                fz = ss * f0 + cc * f1
                fax[sl] = fx
                faz[sl] = fz
                fat[sl] = -f2
                fbx[sl] = -fx
                fbz[sl] = -fz
                fbt[sl] = -f5

        A = unpack(setA)
        B = unpack(setB)

        def stage(s, X, Y):
            wait_elem_loads(X)
            wait_gathers(X)

            @pl.when(s + 1 < i32(nchunk))
            def _():
                @pl.when(s >= 1)
                def _():
                    wait_scatters(Y)
                start_idx_loads(Y, s + 1)
                start_elem_loads(Y, s + 1)
                wait_idx_loads(Y)
                start_gathers(Y)
            compute(X)
            start_scatters(X)

        start_idx_loads(A, jnp.int32(0))
        start_elem_loads(A, jnp.int32(0))
        wait_idx_loads(A)
        start_gathers(A)

        def step(s, _):
            @pl.when(s % 2 == 0)
            def _():
                stage(s, A, B)

            @pl.when(s % 2 == 1)
            def _():
                stage(s, B, A)
            return 0
        lax.fori_loop(jnp.int32(0), jnp.int32(nchunk), step, 0)

        wait_scatters(A if nchunk % 2 == 1 else B)
        wait_scatters(B if nchunk % 2 == 1 else A)

        plsc.subcore_barrier()
        obase = cid * i32(3 * accn) + sid * i32(npt)
        for j, accr in enumerate((ax, az, at)):
            pltpu.sync_copy(accr.at[sl0], zv)
            pltpu.sync_copy(zv, out.at[pl.ds(obase + i32(j * accn), npt)])

    mesh = plsc.VectorSubcoreMesh(
        core_axis_name="c", subcore_axis_name="s",
        num_cores=NC, num_subcores=NS)
    cvec = lambda dt: pltpu.VMEM((CHUNK,), dt)
    bufset = ([cvec(jnp.int32)] * 2 + [cvec(jnp.float32)] * 18
              + [pltpu.SemaphoreType.DMA] * 4)
    return pl.kernel(
        body,
        out_type=jax.ShapeDtypeStruct((NC * 3 * accn,), jnp.float32),
        mesh=mesh,
        scratch_types=(
            bufset + bufset
            + [pltpu.VMEM((16,), jnp.float32)] * 2
            + [pltpu.VMEM((npt,), jnp.float32)]
            + [pltpu.VMEM_SHARED((accn,), jnp.float32)] * 3
        ),
    )


def _loss_tc(fint_r, fe_r, m_r):
    half = fint_r.shape[0] // 2

    def body(fi_ref, fe_ref, m_ref, o_ref):
        f = fi_ref[:half, :] + fi_ref[half:, :]
        fe = fe_ref[...]
        m = m_ref[...]
        r = (f - fe) * m
        ff = fe * m
        num = jnp.sum(r * r)
        den = jnp.maximum(jnp.sum(ff * ff), 1e-30)
        o_ref[...] = jnp.reshape(num / den, (1, 1))

    return pl.pallas_call(
        body,
        out_shape=jax.ShapeDtypeStruct((1, 1), jnp.float32),
    )(fint_r, fe_r, m_r)


def kernel(pred_raw, u_c, theta_c, connectivity, elem_lengths, prop_E,
           prop_A, prop_I22, elem_directions, F_ext, bc_disp, bc_rot):
    f32 = jnp.float32
    n_nodes = pred_raw.shape[0]
    n_elem = connectivity.shape[0]
    npt = -(-n_nodes // NS // 128) * 128
    accn = NS * npt

    conn = connectivity.astype(jnp.int32)
    na = conn[:, 0]
    nb = conn[:, 1]
    cs = elem_directions[:, 0].astype(f32)
    sn = elem_directions[:, 2].astype(f32)
    pr = pred_raw.astype(f32)
    ux, uz, uth = pr[:, 0], pr[:, 1], pr[:, 2]
    uc16 = jnp.broadcast_to(u_c.astype(f32), (16,))
    th16 = jnp.broadcast_to(theta_c.astype(f32), (16,))
    zsrc = jnp.zeros((npt,), f32)

    fint2 = _sc_forces(n_nodes, n_elem)(
        ux, uz, uth, na, nb, elem_lengths.astype(f32), prop_E.astype(f32),
        prop_A.astype(f32), prop_I22.astype(f32), cs, sn, uc16, th16, zsrc)

    pad = ((0, 0), (0, accn - n_nodes))
    fe_cm = jnp.pad(F_ext.astype(f32).T, pad)
    free_d = 1.0 - bc_disp.astype(f32)[:, 0]
    free_r = 1.0 - bc_rot.astype(f32)[:, 0]
    m_cm = jnp.pad(jnp.stack([free_d, free_d, free_r]), pad)

    rows = 3 * accn // 128
    loss = _loss_tc(fint2.reshape(2 * rows, 128),
                    fe_cm.reshape(rows, 128), m_cm.reshape(rows, 128))
    return loss[0, 0].astype(f32)
